# baseline (speedup 1.0000x reference)
import functools

import jax
import jax.numpy as jnp
from jax import lax
from jax.experimental import pallas as pl
from jax.experimental.pallas import tpu as pltpu
from jax.experimental.pallas import tpu_sc as plsc

N = 10000
E = 320000
D = 128
NC = 2
NS = 16
L = 16
NW = NC * NS
EPW = E // NW
B = 64
K = 160
E_PAD = NW * K * B
NCHUNK = K
N_PAD = 10240
RPT = N_PAD // NS


def _matmul_body(x_ref, w_ref, b_ref, o_ref):
    o_ref[...] = (
        jnp.dot(x_ref[...], w_ref[...], preferred_element_type=jnp.float32)
        + b_ref[...]
    )


def _value_proj(x, w, b):
    return pl.pallas_call(
        _matmul_body,
        out_shape=jax.ShapeDtypeStruct((N, D), jnp.float32),
    )(x, w, b)


NBUF = 5
NOUTER = NCHUNK // NBUF
GLEAD = 4


def _sc_body(v_hbm, src_hbm, dst_hbm, att_hbm, out_hbm,
             rows0, rows1, rows2, rows3, rows4,
             didx0, didx1, didx2, didx3, didx4,
             sidx0, sidx1, sidx2, sidx3, sidx4,
             att0, att1, att2, att3, att4,
             acc_sh,
             semg0, semg1, semg2, semg3, semg4,
             semi0, semi1, semi2, semi3, semi4,
             sems0, sems1, sems2, sems3, sems4,
             semsi0, semsi1, semsi2, semsi3, semsi4):
    rows = [rows0, rows1, rows2, rows3, rows4]
    didx = [didx0, didx1, didx2, didx3, didx4]
    sidx = [sidx0, sidx1, sidx2, sidx3, sidx4]
    attb = [att0, att1, att2, att3, att4]
    semg = [semg0, semg1, semg2, semg3, semg4]
    semi = [semi0, semi1, semi2, semi3, semi4]
    sems = [sems0, sems1, sems2, sems3, sems4]
    semsi = [semsi0, semsi1, semsi2, semsi3, semsi4]
    cid = lax.axis_index("c")
    sid = lax.axis_index("s")
    wid = sid * NC + cid
    tile_base = sid * RPT

    def _fire_idx(b, ci):
        pltpu.async_copy(dst_hbm.at[wid, ci], didx[b], semi[b])
        pltpu.async_copy(att_hbm.at[wid, ci], attb[b], semi[b])

    def _wait_idx(b, ci):
        pltpu.make_async_copy(dst_hbm.at[wid, ci], didx[b], semi[b]).wait()
        pltpu.make_async_copy(att_hbm.at[wid, ci], attb[b], semi[b]).wait()

    def _fire_sidx(b, ci):
        pltpu.async_copy(src_hbm.at[wid, ci], sidx[b], semsi[b])

    def _wait_sidx(b, ci):
        pltpu.make_async_copy(src_hbm.at[wid, ci], sidx[b], semsi[b]).wait()

    def _fire_gather(b, ci):
        pltpu.async_copy(v_hbm.at[didx[b]], rows[b], semg[b])

    def _wait_gather(b, ci):
        pltpu.make_async_copy(v_hbm.at[didx[b]], rows[b], semg[b]).wait()

    def _fire_scatter(b):
        pltpu.async_copy(rows[b], acc_sh.at[sidx[b]], sems[b], add=True)

    def _wait_scatter(b):
        pltpu.make_async_copy(rows[b], acc_sh.at[sidx[b]], sems[b]).wait()

    def _zero_row(i, carry):
        for j in range(D // L):
            rows0[i, pl.ds(j * L, L)] = jnp.zeros((L,), jnp.float32)
        return carry

    lax.fori_loop(0, B, _zero_row, 0)
    for k in range(RPT // B):
        pltpu.sync_copy(rows0, acc_sh.at[pl.ds(tile_base + k * B, B)])

    for b in range(NBUF):
        _fire_idx(b, b)
        _fire_sidx(b, b)
    for b in range(GLEAD):
        _wait_idx(b, b)
        _fire_gather(b, b)

    plsc.subcore_barrier()

    def _scale(rbuf, av):
        def _scale_group(g, c2):
            av16 = av[pl.ds(g * L, L)]
            for lane in range(L):
                a = lax.gather(
                    av16, jnp.full((L, 1), lane, jnp.int32),
                    lax.GatherDimensionNumbers(
                        offset_dims=(), collapsed_slice_dims=(0,),
                        start_index_map=(0,)),
                    slice_sizes=(1,),
                    mode=lax.GatherScatterMode.PROMISE_IN_BOUNDS)
                e = g * L + lane
                for j in range(D // L):
                    rbuf[e, pl.ds(j * L, L)] = rbuf[e, pl.ds(j * L, L)] * a
            return c2

        lax.fori_loop(0, B // L, _scale_group, 0)

    def _outer(g, carry):
        for b in range(NBUF):
            ci = g * NBUF + b

            @pl.when(ci < NCHUNK)
            def _consume():
                _wait_gather(b, ci)
                _wait_sidx(b, ci)
                _scale(rows[b], attb[b])
                _fire_scatter(b)

            @pl.when(ci + NBUF < NCHUNK)
            def _refill():
                _fire_idx(b, ci + NBUF)

            b2 = (b + GLEAD) % NBUF

            @pl.when(ci + GLEAD < NCHUNK)
            def _gather_ahead():
                @pl.when(ci + GLEAD >= NBUF)
                def _drain_prev():
                    _wait_scatter(b2)
                    _fire_sidx(b2, ci + GLEAD)

                _wait_idx(b2, ci + GLEAD)
                _fire_gather(b2, ci + GLEAD)
        return carry

    lax.fori_loop(0, NOUTER, _outer, 0)

    for b in range(NBUF):
        _wait_scatter(b)

    plsc.subcore_barrier()

    pltpu.sync_copy(
        acc_sh.at[pl.ds(tile_base, RPT)],
        out_hbm.at[cid, pl.ds(tile_base, RPT)],
    )


_sc_agg = functools.partial(
    pl.kernel,
    out_type=jax.ShapeDtypeStruct((NC, N_PAD, D), jnp.float32),
    mesh=plsc.VectorSubcoreMesh(
        core_axis_name="c", subcore_axis_name="s",
        num_cores=NC, num_subcores=NS,
    ),
    scratch_types=(
        [pltpu.VMEM((B, D), jnp.float32)] * NBUF
        + [pltpu.VMEM((B,), jnp.int32)] * NBUF
        + [pltpu.VMEM((B,), jnp.int32)] * NBUF
        + [pltpu.VMEM((B,), jnp.float32)] * NBUF
        + [pltpu.VMEM_SHARED((N_PAD, D), jnp.float32)]
        + [pltpu.SemaphoreType.DMA] * (4 * NBUF)
    ),
)(_sc_body)


def _bn_body(p_ref, g_ref, b_ref, o_ref):
    s = p_ref[0, :N, :] + p_ref[1, :N, :]
    mean = jnp.mean(s, axis=0, keepdims=True)
    var = jnp.mean(jnp.square(s - mean), axis=0, keepdims=True)
    o = (s - mean) * jax.lax.rsqrt(var + 1e-5) * g_ref[...] + b_ref[...]
    o_ref[...] = jnp.where(o >= 0, o, 0.01 * o)


def _bn_leaky(parts, gamma, beta):
    return pl.pallas_call(
        _bn_body,
        out_shape=jax.ShapeDtypeStruct((N, D), jnp.float32),
    )(parts, gamma, beta)


def kernel(x, src, dst, att_score, Wv, bv, gamma, beta):
    w_cat = jnp.transpose(Wv, (1, 0, 2)).reshape(D, D)
    b_cat = bv.reshape(1, D)
    v = _value_proj(x, w_cat, b_cat)
    zpad = jnp.zeros((E_PAD - E,), jnp.int32)
    src_p = jnp.concatenate([src, zpad])
    dst_p = jnp.concatenate([dst, zpad])
    att_p = jnp.concatenate(
        [att_score.reshape(E), jnp.zeros((E_PAD - E,), jnp.float32)])
    parts = _sc_agg(
        v,
        src_p.reshape(NW, NCHUNK, B),
        dst_p.reshape(NW, NCHUNK, B),
        att_p.reshape(NW, NCHUNK, B),
    )
    return _bn_leaky(parts, gamma.reshape(1, D), beta.reshape(1, D))


# speedup vs baseline: 3.1989x; 3.1989x over previous
import functools

import jax
import jax.numpy as jnp
from jax import lax
from jax.experimental import pallas as pl
from jax.experimental.pallas import tpu as pltpu
from jax.experimental.pallas import tpu_sc as plsc

N = 10000
E = 320000
D = 128
NC = 2
NS = 16
L = 16
NW = NC * NS
EPW = E // NW
B = 80
NCHUNK = EPW // B
N_PAD = 10240
RPT = N_PAD // NS


def _matmul_body(x_ref, w_ref, b_ref, o_ref):
    o_ref[...] = (
        jnp.dot(x_ref[...], w_ref[...], preferred_element_type=jnp.float32)
        + b_ref[...]
    )


def _value_proj(x, w, b):
    return pl.pallas_call(
        _matmul_body,
        out_shape=jax.ShapeDtypeStruct((N, D), jnp.float32),
    )(x, w, b)


NBUF = 4
NOUTER = 31
GLEAD = 3


def _sc_body(v_hbm, src_hbm, dst_hbm, att_hbm, out_hbm,
             rows0, rows1, rows2, rows3,
             didx0, didx1, didx2, didx3,
             sidx0, sidx1, sidx2, sidx3,
             att0, att1, att2, att3,
             acc_sh,
             semg0, semg1, semg2, semg3,
             semi0, semi1, semi2, semi3,
             sems0, sems1, sems2, sems3,
             semsi0, semsi1, semsi2, semsi3):
    rows = [rows0, rows1, rows2, rows3]
    didx = [didx0, didx1, didx2, didx3]
    sidx = [sidx0, sidx1, sidx2, sidx3]
    attb = [att0, att1, att2, att3]
    semg = [semg0, semg1, semg2, semg3]
    semi = [semi0, semi1, semi2, semi3]
    sems = [sems0, sems1, sems2, sems3]
    semsi = [semsi0, semsi1, semsi2, semsi3]
    cid = lax.axis_index("c")
    sid = lax.axis_index("s")
    wid = sid * NC + cid
    tile_base = sid * RPT

    def _fire_idx(b, ci):
        pltpu.async_copy(dst_hbm.at[wid, ci], didx[b], semi[b])
        pltpu.async_copy(att_hbm.at[wid, ci], attb[b], semi[b])

    def _wait_idx(b, ci):
        pltpu.make_async_copy(dst_hbm.at[wid, ci], didx[b], semi[b]).wait()
        pltpu.make_async_copy(att_hbm.at[wid, ci], attb[b], semi[b]).wait()

    def _fire_sidx(b, ci):
        pltpu.async_copy(src_hbm.at[wid, ci], sidx[b], semsi[b])

    def _wait_sidx(b, ci):
        pltpu.make_async_copy(src_hbm.at[wid, ci], sidx[b], semsi[b]).wait()

    def _fire_gather(b, ci):
        pltpu.async_copy(v_hbm.at[didx[b]], rows[b], semg[b])

    def _wait_gather(b, ci):
        pltpu.make_async_copy(v_hbm.at[didx[b]], rows[b], semg[b]).wait()

    def _fire_scatter(b):
        pltpu.async_copy(rows[b], acc_sh.at[sidx[b]], sems[b], add=True)

    def _wait_scatter(b):
        pltpu.make_async_copy(rows[b], acc_sh.at[sidx[b]], sems[b]).wait()

    def _zero_row(i, carry):
        for j in range(D // L):
            rows0[i, pl.ds(j * L, L)] = jnp.zeros((L,), jnp.float32)
        return carry

    lax.fori_loop(0, B, _zero_row, 0)
    for k in range(RPT // B):
        pltpu.async_copy(rows0, acc_sh.at[pl.ds(tile_base + k * B, B)], semg0)
    for k in range(RPT // B):
        pltpu.make_async_copy(
            rows0, acc_sh.at[pl.ds(tile_base + k * B, B)], semg0).wait()

    for b in range(NBUF):
        _fire_idx(b, b)
        _fire_sidx(b, b)
    for b in range(GLEAD):
        _wait_idx(b, b)
        _fire_gather(b, b)

    plsc.subcore_barrier()

    def _scale(rbuf, av):
        def _scale_group(g, c2):
            av16 = av[pl.ds(g * L, L)]
            for lane in range(L):
                a = lax.gather(
                    av16, jnp.full((L, 1), lane, jnp.int32),
                    lax.GatherDimensionNumbers(
                        offset_dims=(), collapsed_slice_dims=(0,),
                        start_index_map=(0,)),
                    slice_sizes=(1,),
                    mode=lax.GatherScatterMode.PROMISE_IN_BOUNDS)
                e = g * L + lane
                for j in range(D // L):
                    rbuf[e, pl.ds(j * L, L)] = rbuf[e, pl.ds(j * L, L)] * a
            return c2

        lax.fori_loop(0, B // L, _scale_group, 0)

    def _outer(g, carry):
        for b in range(NBUF):
            ci = g * NBUF + b

            @pl.when(ci < NCHUNK)
            def _consume():
                _wait_gather(b, ci)
                _wait_sidx(b, ci)
                _scale(rows[b], attb[b])
                _fire_scatter(b)

            @pl.when(ci + NBUF < NCHUNK)
            def _refill():
                _fire_idx(b, ci + NBUF)

            b2 = (b + GLEAD) % NBUF

            @pl.when(ci + GLEAD < NCHUNK)
            def _gather_ahead():
                @pl.when(ci + GLEAD >= NBUF)
                def _drain_prev():
                    _wait_scatter(b2)
                    _fire_sidx(b2, ci + GLEAD)

                _wait_idx(b2, ci + GLEAD)
                _fire_gather(b2, ci + GLEAD)
        return carry

    lax.fori_loop(0, NOUTER + 1, _outer, 0)

    for b in (1, 2, 3, 0):
        _wait_scatter(b)

    plsc.subcore_barrier()

    pltpu.sync_copy(
        acc_sh.at[pl.ds(tile_base, RPT)],
        out_hbm.at[cid, pl.ds(tile_base, RPT)],
    )


_sc_agg = functools.partial(
    pl.kernel,
    out_type=jax.ShapeDtypeStruct((NC, N_PAD, D), jnp.float32),
    mesh=plsc.VectorSubcoreMesh(
        core_axis_name="c", subcore_axis_name="s",
        num_cores=NC, num_subcores=NS,
    ),
    scratch_types=(
        [pltpu.VMEM((B, D), jnp.float32)] * NBUF
        + [pltpu.VMEM((B,), jnp.int32)] * NBUF
        + [pltpu.VMEM((B,), jnp.int32)] * NBUF
        + [pltpu.VMEM((B,), jnp.float32)] * NBUF
        + [pltpu.VMEM_SHARED((N_PAD, D), jnp.float32)]
        + [pltpu.SemaphoreType.DMA] * (4 * NBUF)
    ),
)(_sc_body)


def _bn_body(p_ref, g_ref, b_ref, o_ref):
    s = p_ref[0, :N, :] + p_ref[1, :N, :]
    mean = jnp.mean(s, axis=0, keepdims=True)
    var = jnp.mean(jnp.square(s - mean), axis=0, keepdims=True)
    o = (s - mean) * jax.lax.rsqrt(var + 1e-5) * g_ref[...] + b_ref[...]
    o_ref[...] = jnp.where(o >= 0, o, 0.01 * o)


def _bn_leaky(parts, gamma, beta):
    return pl.pallas_call(
        _bn_body,
        out_shape=jax.ShapeDtypeStruct((N, D), jnp.float32),
    )(parts, gamma, beta)


def kernel(x, src, dst, att_score, Wv, bv, gamma, beta):
    w_cat = jnp.transpose(Wv, (1, 0, 2)).reshape(D, D)
    b_cat = bv.reshape(1, D)
    v = _value_proj(x, w_cat, b_cat)
    parts = _sc_agg(
        v,
        src.reshape(NW, NCHUNK, B),
        dst.reshape(NW, NCHUNK, B),
        att_score.reshape(NW, NCHUNK, B),
    )
    return _bn_leaky(parts, gamma.reshape(1, D), beta.reshape(1, D))
